# Initial kernel scaffold; baseline (speedup 1.0000x reference)
import jax
import jax.numpy as jnp
from jax import lax
from jax.experimental import pallas as pl
from jax.experimental.pallas import tpu as pltpu
from jax.experimental.pallas import tpu_sc as plsc

N_USERS = 10000
N_BOOKS = 40000
N = N_USERS + N_BOOKS
E = 800000
D_IN = 128
H = 64
HH = 32

NC = 2
NS = 16

CHUNKS = 49
ROWS_PER_TILE = CHUNKS * 8
ROWS_TOTAL = ROWS_PER_TILE * NS
E_PAD = ROWS_TOTAL * 128
NP = N + 8
RPT = N // NS

BLK = 2000
GRID = N // BLK
f32 = jnp.float32


def _make_sc_agg(with_cnt):
  mesh = plsc.VectorSubcoreMesh(
      core_axis_name="c", subcore_axis_name="s", num_cores=NC, num_subcores=NS)
  out_type = [jax.ShapeDtypeStruct((N, HH), f32),
              jax.ShapeDtypeStruct((N, HH), f32)]
  if with_cnt:
    out_type += [jax.ShapeDtypeStruct((N, 8), f32),
                 jax.ShapeDtypeStruct((N, 8), f32)]
  scratch = [
      pltpu.VMEM((8, 128), jnp.int32),
      pltpu.VMEM((8, 128), jnp.int32),
      pltpu.VMEM((8, 128, HH), f32),
      pltpu.VMEM((128, 8), f32),
      pltpu.VMEM_SHARED((NP, HH), f32),
      pltpu.VMEM_SHARED((NP, 8), f32),
      pltpu.SemaphoreType.DMA,
  ]

  def body(x0, x1, graph3, z32, z8, ones_hbm, *rest):
    if with_cnt:
      agg0, agg1, cnt0, cnt1 = rest[:4]
      src8, dst8, rows, ones_v, acc, cacc, sem = rest[4:]
    else:
      agg0, agg1 = rest[:2]
      src8, dst8, rows, ones_v, acc, cacc, sem = rest[2:]
    c = lax.axis_index("c")
    s = lax.axis_index("s")

    r0 = s * RPT
    pltpu.sync_copy(z32.at[pl.ds(r0, RPT), :], acc.at[pl.ds(r0, RPT), :])
    if with_cnt:
      pltpu.sync_copy(z8.at[pl.ds(r0, RPT), :], cacc.at[pl.ds(r0, RPT), :])
      pltpu.sync_copy(ones_hbm, ones_v)

    @pl.when(s == NS - 1)
    def _zero_trash():
      pltpu.sync_copy(z32.at[pl.ds(N, 8), :], acc.at[pl.ds(N, 8), :])
      if with_cnt:
        pltpu.sync_copy(z8.at[pl.ds(N, 8), :], cacc.at[pl.ds(N, 8), :])

    plsc.subcore_barrier()

    def run(xc, parity):
      def chunk(kc, carry):
        base = s * ROWS_PER_TILE + kc * 8
        pltpu.sync_copy(graph3.at[0, pl.ds(base, 8), :], src8)
        pltpu.sync_copy(graph3.at[1, pl.ds(base, 8), :], dst8)
        descs = [pltpu.async_copy(xc.at[src8.at[j]], rows.at[j], sem)
                 for j in range(8)]
        for d in descs:
          d.wait()
        for j in range(8):
          pltpu.sync_copy(rows.at[j], acc.at[dst8.at[j]], add=True)
        if with_cnt:
          @pl.when((kc % 2) == parity)
          def _cnt():
            for j in range(8):
              pltpu.sync_copy(ones_v, cacc.at[dst8.at[j]], add=True)
        return carry
      lax.fori_loop(0, CHUNKS, chunk, 0)

    @pl.when(c == 0)
    def _run0():
      run(x0, 0)

    @pl.when(c == 1)
    def _run1():
      run(x1, 1)

    plsc.subcore_barrier()

    @pl.when(c == 0)
    def _out0():
      pltpu.sync_copy(acc.at[pl.ds(r0, RPT), :], agg0.at[pl.ds(r0, RPT), :])
      if with_cnt:
        pltpu.sync_copy(cacc.at[pl.ds(r0, RPT), :], cnt0.at[pl.ds(r0, RPT), :])

    @pl.when(c == 1)
    def _out1():
      pltpu.sync_copy(acc.at[pl.ds(r0, RPT), :], agg1.at[pl.ds(r0, RPT), :])
      if with_cnt:
        pltpu.sync_copy(cacc.at[pl.ds(r0, RPT), :], cnt1.at[pl.ds(r0, RPT), :])

  return pl.kernel(body, out_type=out_type, mesh=mesh, scratch_types=scratch)


_sc_agg_cnt = _make_sc_agg(True)
_sc_agg = _make_sc_agg(False)


def _embed_body(uf, bf, w1u, b1u, w2u, b2u, w1b, b1b, w2b, b2b, o0, o1):
  is_user = pl.program_id(0) < N_USERS // BLK
  f = jnp.where(is_user, uf[...], bf[...])
  w1 = jnp.where(is_user, w1u[...], w1b[...])
  b1 = jnp.where(is_user, b1u[...], b1b[...])
  w2 = jnp.where(is_user, w2u[...], w2b[...])
  b2 = jnp.where(is_user, b2u[...], b2b[...])
  h = jnp.maximum(jnp.dot(f, w1, preferred_element_type=f32) + b1, 0.0)
  e = jnp.dot(h, w2, preferred_element_type=f32) + b2
  o0[...] = e[:, :HH]
  o1[...] = e[:, HH:]


def _embed(uf, bf, w1ut, b1u, w2ut, b2u, w1bt, b1b, w2bt, b2b):
  cmap = lambda i: (0, 0)
  return pl.pallas_call(
      _embed_body,
      grid=(GRID,),
      in_specs=[
          pl.BlockSpec((BLK, D_IN), lambda i: (jnp.minimum(i, N_USERS // BLK - 1), 0)),
          pl.BlockSpec((BLK, D_IN), lambda i: (jnp.maximum(i - N_USERS // BLK, 0), 0)),
          pl.BlockSpec((D_IN, H), cmap), pl.BlockSpec((1, H), cmap),
          pl.BlockSpec((H, H), cmap), pl.BlockSpec((1, H), cmap),
          pl.BlockSpec((D_IN, H), cmap), pl.BlockSpec((1, H), cmap),
          pl.BlockSpec((H, H), cmap), pl.BlockSpec((1, H), cmap),
      ],
      out_specs=[pl.BlockSpec((BLK, HH), lambda i: (i, 0)),
                 pl.BlockSpec((BLK, HH), lambda i: (i, 0))],
      out_shape=[jax.ShapeDtypeStruct((N, HH), f32),
                 jax.ShapeDtypeStruct((N, HH), f32)],
  )(uf, bf, w1ut, b1u, w2ut, b2u, w1bt, b1b, w2bt, b2b)


def _lin_body(a0, a1, c0, c1, x0, x1, wlt, bl, wrt, h_ref, st_ref, acc):
  i = pl.program_id(0)
  agg = jnp.concatenate([a0[...], a1[...]], axis=1)
  cnt = c0[...][:, :1] + c1[...][:, :1]
  rc = 1.0 / jnp.maximum(cnt, 1.0)
  x = jnp.concatenate([x0[...], x1[...]], axis=1)
  h = (jnp.dot(agg * rc, wlt[...], preferred_element_type=f32) + bl[...]
       + jnp.dot(x, wrt[...], preferred_element_type=f32))
  h_ref[...] = h
  st = jnp.stack([jnp.sum(h, axis=0), jnp.sum(h * h, axis=0)])

  @pl.when(i == 0)
  def _init():
    acc[...] = st

  @pl.when(i > 0)
  def _accum():
    acc[...] += st

  st_ref[...] = acc[...]


def _lin(agg0, agg1, cnt0, cnt1, x0, x1, wlt, bl, wrt):
  cmap = lambda i: (0, 0)
  return pl.pallas_call(
      _lin_body,
      grid=(GRID,),
      in_specs=[
          pl.BlockSpec((BLK, HH), lambda i: (i, 0)),
          pl.BlockSpec((BLK, HH), lambda i: (i, 0)),
          pl.BlockSpec((BLK, 8), lambda i: (i, 0)),
          pl.BlockSpec((BLK, 8), lambda i: (i, 0)),
          pl.BlockSpec((BLK, HH), lambda i: (i, 0)),
          pl.BlockSpec((BLK, HH), lambda i: (i, 0)),
          pl.BlockSpec((H, H), cmap), pl.BlockSpec((1, H), cmap),
          pl.BlockSpec((H, H), cmap),
      ],
      out_specs=[pl.BlockSpec((BLK, H), lambda i: (i, 0)),
                 pl.BlockSpec((2, H), cmap)],
      out_shape=[jax.ShapeDtypeStruct((N, H), f32),
                 jax.ShapeDtypeStruct((2, H), f32)],
      scratch_shapes=[pltpu.VMEM((2, H), f32)],
  )(agg0, agg1, cnt0, cnt1, x0, x1, wlt, bl, wrt)


def _bn_body_split(h_ref, st_ref, g_ref, b_ref, o0, o1):
  st = st_ref[...]
  m = st[0] / N
  v = st[1] / N - m * m
  sc = g_ref[0] * lax.rsqrt(v + 1e-5)
  y = jnp.maximum((h_ref[...] - m) * sc + b_ref[0], 0.0)
  o0[...] = y[:, :HH]
  o1[...] = y[:, HH:]


def _bn_body_full(h_ref, st_ref, g_ref, b_ref, o_ref):
  st = st_ref[...]
  m = st[0] / N
  v = st[1] / N - m * m
  sc = g_ref[0] * lax.rsqrt(v + 1e-5)
  o_ref[...] = jnp.maximum((h_ref[...] - m) * sc + b_ref[0], 0.0)


def _bn(h, st, g, b, split):
  cmap = lambda i: (0, 0)
  in_specs = [
      pl.BlockSpec((BLK, H), lambda i: (i, 0)),
      pl.BlockSpec((2, H), cmap),
      pl.BlockSpec((1, H), cmap), pl.BlockSpec((1, H), cmap),
  ]
  if split:
    return pl.pallas_call(
        _bn_body_split, grid=(GRID,), in_specs=in_specs,
        out_specs=[pl.BlockSpec((BLK, HH), lambda i: (i, 0)),
                   pl.BlockSpec((BLK, HH), lambda i: (i, 0))],
        out_shape=[jax.ShapeDtypeStruct((N, HH), f32),
                   jax.ShapeDtypeStruct((N, HH), f32)],
    )(h, st, g, b)
  return pl.pallas_call(
      _bn_body_full, grid=(GRID,), in_specs=in_specs,
      out_specs=pl.BlockSpec((BLK, H), lambda i: (i, 0)),
      out_shape=jax.ShapeDtypeStruct((N, H), f32),
  )(h, st, g, b)


def kernel(graph_data, user_features, book_features, W1u, b1u, W2u, b2u,
           W1b, b1b, W2b, b2b, Wl1, bl1, Wr1, Wl2, bl2, Wr2,
           g1, beta1, g2, beta2):
  npad = E_PAD - E
  pad_src = (jnp.arange(npad, dtype=jnp.int32) * 17) % N
  pad_dst = jnp.full((npad,), N, dtype=jnp.int32)
  graph3 = jnp.concatenate(
      [graph_data, jnp.stack([pad_src, pad_dst])], axis=1
  ).reshape(2, ROWS_TOTAL, 128)

  z32 = jnp.zeros((NP, HH), f32)
  z8 = jnp.zeros((NP, 8), f32)
  ones8 = jnp.ones((128, 8), f32)

  r2 = lambda a: a.reshape(1, H)
  x0, x1 = _embed(user_features, book_features,
                  W1u.T, r2(b1u), W2u.T, r2(b2u),
                  W1b.T, r2(b1b), W2b.T, r2(b2b))

  agg0, agg1, cnt0, cnt1 = _sc_agg_cnt(x0, x1, graph3, z32, z8, ones8)
  h1, st1 = _lin(agg0, agg1, cnt0, cnt1, x0, x1, Wl1.T, r2(bl1), Wr1.T)
  x0, x1 = _bn(h1, st1, r2(g1), r2(beta1), split=True)

  agg0, agg1 = _sc_agg(x0, x1, graph3, z32, z8, ones8)
  h2, st2 = _lin(agg0, agg1, cnt0, cnt1, x0, x1, Wl2.T, r2(bl2), Wr2.T)
  return _bn(h2, st2, r2(g2), r2(beta2), split=False)


# speedup vs baseline: 5.1056x; 5.1056x over previous
import functools

import jax
import jax.numpy as jnp
from jax import lax
from jax.experimental import pallas as pl
from jax.experimental.pallas import tpu as pltpu
from jax.experimental.pallas import tpu_sc as plsc

N_USERS = 10000
N_BOOKS = 40000
N = N_USERS + N_BOOKS
E = 800000
D_IN = 128
H = 64
Q = 16

NC = 2
NS = 16

ROWS = E // 128
FULL = 48
RQT = N // NS
CNT_Q = 3128

BLK = 2000
GRID = N // BLK
f32 = jnp.float32


def _make_sc_agg(with_cnt):
  mesh = plsc.VectorSubcoreMesh(
      core_axis_name="c", subcore_axis_name="s", num_cores=NC, num_subcores=NS)
  out_type = [jax.ShapeDtypeStruct((N, Q), f32) for _ in range(4)]
  if with_cnt:
    out_type += [jax.ShapeDtypeStruct((N,), f32),
                 jax.ShapeDtypeStruct((N,), f32)]
  scratch = [
      pltpu.VMEM((8, 128), jnp.int32),
      pltpu.VMEM((8, 128), jnp.int32),
      pltpu.VMEM((8, 128, Q), f32),
      pltpu.VMEM((128,), f32),
      pltpu.VMEM_SHARED((N, Q), f32),
      pltpu.VMEM_SHARED((N,), f32),
      pltpu.SemaphoreType.DMA,
  ]
  CNT_L = N - (NS - 1) * CNT_Q

  def body(x00, x01, x10, x11, graph3, z16, z1, ones_hbm, *rest):
    if with_cnt:
      aggs = rest[:4]
      cnt0, cnt1 = rest[4:6]
      src8, dst8, rows, ones_v, acc, cacc, sem = rest[6:]
    else:
      aggs = rest[:4]
      src8, dst8, rows, ones_v, acc, cacc, sem = rest[4:]
    c = lax.axis_index("c")
    s = lax.axis_index("s")
    my0 = s * RQT
    row0 = s * 390 + jnp.minimum(s, 10)
    tail = jnp.where(s < 10, 7, 6)

    def zero_acc():
      pltpu.sync_copy(z16.at[pl.ds(my0, RQT), :], acc.at[pl.ds(my0, RQT), :])

    def cnt_slices(a, b):
      @pl.when(s < NS - 1)
      def _a():
        pltpu.sync_copy(a.at[pl.ds(s * CNT_Q, CNT_Q)],
                        b.at[pl.ds(s * CNT_Q, CNT_Q)])

      @pl.when(s == NS - 1)
      def _b():
        pltpu.sync_copy(a.at[pl.ds((NS - 1) * CNT_Q, CNT_L)],
                        b.at[pl.ds((NS - 1) * CNT_Q, CNT_L)])

    zero_acc()
    if with_cnt:
      cnt_slices(z1, cacc)
      pltpu.sync_copy(ones_hbm, ones_v)
    plsc.subcore_barrier()

    def scatter_all(xq, parity, do_cnt):
      def chunk(kc, carry):
        base = row0 + kc * 8
        pltpu.sync_copy(graph3.at[0, pl.ds(base, 8), :], src8)
        pltpu.sync_copy(graph3.at[1, pl.ds(base, 8), :], dst8)
        descs = [pltpu.async_copy(xq.at[src8.at[j]], rows.at[j], sem)
                 for j in range(8)]
        for d in descs:
          d.wait()
        for j in range(8):
          pltpu.sync_copy(rows.at[j], acc.at[dst8.at[j]], add=True)
        if do_cnt:
          @pl.when((kc % 2) == parity)
          def _cnt():
            for j in range(8):
              pltpu.sync_copy(ones_v, cacc.at[dst8.at[j]], add=True)
        return carry
      lax.fori_loop(0, FULL, chunk, 0)

      def tailrow(j, carry):
        base = row0 + FULL * 8 + j
        pltpu.sync_copy(graph3.at[0, pl.ds(base, 1), :], src8.at[pl.ds(0, 1), :])
        pltpu.sync_copy(graph3.at[1, pl.ds(base, 1), :], dst8.at[pl.ds(0, 1), :])
        pltpu.async_copy(xq.at[src8.at[0]], rows.at[0], sem).wait()
        pltpu.sync_copy(rows.at[0], acc.at[dst8.at[0]], add=True)
        if do_cnt:
          @pl.when(parity == 0)
          def _cnt():
            pltpu.sync_copy(ones_v, cacc.at[dst8.at[0]], add=True)
        return carry
      lax.fori_loop(0, tail, tailrow, 0)

    for p in range(2):
      do_cnt = with_cnt and p == 0

      @pl.when(c == 0)
      def _run0():
        scatter_all(x00 if p == 0 else x01, 0, do_cnt)

      @pl.when(c == 1)
      def _run1():
        scatter_all(x10 if p == 0 else x11, 1, do_cnt)

      plsc.subcore_barrier()

      @pl.when(c == 0)
      def _out0():
        pltpu.sync_copy(acc.at[pl.ds(my0, RQT), :],
                        aggs[p].at[pl.ds(my0, RQT), :])

      @pl.when(c == 1)
      def _out1():
        pltpu.sync_copy(acc.at[pl.ds(my0, RQT), :],
                        aggs[2 + p].at[pl.ds(my0, RQT), :])

      if do_cnt:
        @pl.when(c == 0)
        def _outc0():
          cnt_slices(cacc, cnt0)

        @pl.when(c == 1)
        def _outc1():
          cnt_slices(cacc, cnt1)

      if p == 0:
        zero_acc()
        plsc.subcore_barrier()

  return pl.kernel(
      body, out_type=out_type, mesh=mesh, scratch_types=scratch,
      compiler_params=pltpu.CompilerParams(use_tc_tiling_on_sc=False))


@functools.lru_cache(maxsize=None)
def _get_sc_agg(with_cnt):
  return _make_sc_agg(with_cnt)


def _embed_body(uf, bf, w1u, b1u, w2u, b2u, w1b, b1b, w2b, b2b,
                o0, o1, o2, o3):
  is_user = pl.program_id(0) < N_USERS // BLK
  f = jnp.where(is_user, uf[...], bf[...])
  w1 = jnp.where(is_user, w1u[...], w1b[...])
  b1 = jnp.where(is_user, b1u[...], b1b[...])
  w2 = jnp.where(is_user, w2u[...], w2b[...])
  b2 = jnp.where(is_user, b2u[...], b2b[...])
  h = jnp.maximum(jnp.dot(f, w1, preferred_element_type=f32) + b1, 0.0)
  e = jnp.dot(h, w2, preferred_element_type=f32) + b2
  for k, o in enumerate((o0, o1, o2, o3)):
    o[...] = e[:, k * Q:(k + 1) * Q]


def _embed(uf, bf, w1ut, b1u, w2ut, b2u, w1bt, b1b, w2bt, b2b):
  cmap = lambda i: (0, 0)
  qspec = pl.BlockSpec((BLK, Q), lambda i: (i, 0))
  return pl.pallas_call(
      _embed_body,
      grid=(GRID,),
      in_specs=[
          pl.BlockSpec((BLK, D_IN), lambda i: (jnp.minimum(i, N_USERS // BLK - 1), 0)),
          pl.BlockSpec((BLK, D_IN), lambda i: (jnp.maximum(i - N_USERS // BLK, 0), 0)),
          pl.BlockSpec((D_IN, H), cmap), pl.BlockSpec((1, H), cmap),
          pl.BlockSpec((H, H), cmap), pl.BlockSpec((1, H), cmap),
          pl.BlockSpec((D_IN, H), cmap), pl.BlockSpec((1, H), cmap),
          pl.BlockSpec((H, H), cmap), pl.BlockSpec((1, H), cmap),
      ],
      out_specs=[qspec] * 4,
      out_shape=[jax.ShapeDtypeStruct((N, Q), f32)] * 4,
  )(uf, bf, w1ut, b1u, w2ut, b2u, w1bt, b1b, w2bt, b2b)


def _lin_body(a0, a1, a2, a3, c0, c1, x0, x1, x2, x3, wlt, bl, wrt,
              h_ref, st_ref, acc):
  i = pl.program_id(0)
  agg = jnp.concatenate([a0[...], a1[...], a2[...], a3[...]], axis=1)
  cnt = c0[...] + c1[...]
  rc = 1.0 / jnp.maximum(cnt, 1.0)
  x = jnp.concatenate([x0[...], x1[...], x2[...], x3[...]], axis=1)
  h = (jnp.dot(agg * rc, wlt[...], preferred_element_type=f32) + bl[...]
       + jnp.dot(x, wrt[...], preferred_element_type=f32))
  h_ref[...] = h
  st = jnp.stack([jnp.sum(h, axis=0), jnp.sum(h * h, axis=0)])

  @pl.when(i == 0)
  def _init():
    acc[...] = st

  @pl.when(i > 0)
  def _accum():
    acc[...] += st

  st_ref[...] = acc[...]


def _lin(aggs, cnt0, cnt1, xs, wlt, bl, wrt):
  cmap = lambda i: (0, 0)
  qspec = pl.BlockSpec((BLK, Q), lambda i: (i, 0))
  return pl.pallas_call(
      _lin_body,
      grid=(GRID,),
      in_specs=[qspec] * 4 + [
          pl.BlockSpec((BLK, 1), lambda i: (i, 0)),
          pl.BlockSpec((BLK, 1), lambda i: (i, 0)),
      ] + [qspec] * 4 + [
          pl.BlockSpec((H, H), cmap), pl.BlockSpec((1, H), cmap),
          pl.BlockSpec((H, H), cmap),
      ],
      out_specs=[pl.BlockSpec((BLK, H), lambda i: (i, 0)),
                 pl.BlockSpec((2, H), cmap)],
      out_shape=[jax.ShapeDtypeStruct((N, H), f32),
                 jax.ShapeDtypeStruct((2, H), f32)],
      scratch_shapes=[pltpu.VMEM((2, H), f32)],
  )(*aggs, cnt0, cnt1, *xs, wlt, bl, wrt)


def _bn_body_split(h_ref, st_ref, g_ref, b_ref, o0, o1, o2, o3):
  st = st_ref[...]
  m = st[0] / N
  v = st[1] / N - m * m
  sc = g_ref[0] * lax.rsqrt(v + 1e-5)
  y = jnp.maximum((h_ref[...] - m) * sc + b_ref[0], 0.0)
  for k, o in enumerate((o0, o1, o2, o3)):
    o[...] = y[:, k * Q:(k + 1) * Q]


def _bn_body_full(h_ref, st_ref, g_ref, b_ref, o_ref):
  st = st_ref[...]
  m = st[0] / N
  v = st[1] / N - m * m
  sc = g_ref[0] * lax.rsqrt(v + 1e-5)
  o_ref[...] = jnp.maximum((h_ref[...] - m) * sc + b_ref[0], 0.0)


def _bn(h, st, g, b, split):
  cmap = lambda i: (0, 0)
  in_specs = [
      pl.BlockSpec((BLK, H), lambda i: (i, 0)),
      pl.BlockSpec((2, H), cmap),
      pl.BlockSpec((1, H), cmap), pl.BlockSpec((1, H), cmap),
  ]
  if split:
    qspec = pl.BlockSpec((BLK, Q), lambda i: (i, 0))
    return pl.pallas_call(
        _bn_body_split, grid=(GRID,), in_specs=in_specs,
        out_specs=[qspec] * 4,
        out_shape=[jax.ShapeDtypeStruct((N, Q), f32)] * 4,
    )(h, st, g, b)
  return pl.pallas_call(
      _bn_body_full, grid=(GRID,), in_specs=in_specs,
      out_specs=pl.BlockSpec((BLK, H), lambda i: (i, 0)),
      out_shape=jax.ShapeDtypeStruct((N, H), f32),
  )(h, st, g, b)


def kernel(graph_data, user_features, book_features, W1u, b1u, W2u, b2u,
           W1b, b1b, W2b, b2b, Wl1, bl1, Wr1, Wl2, bl2, Wr2,
           g1, beta1, g2, beta2):
  graph3 = graph_data.reshape(2, ROWS, 128)

  z16 = jnp.zeros((N, Q), f32)
  z1 = jnp.zeros((N,), f32)
  ones128 = jnp.ones((128,), f32)

  r2 = lambda a: a.reshape(1, H)
  xs = _embed(user_features, book_features,
              W1u.T, r2(b1u), W2u.T, r2(b2u),
              W1b.T, r2(b1b), W2b.T, r2(b2b))

  *aggs, cnt0, cnt1 = _get_sc_agg(True)(*xs, graph3, z16, z1, ones128)
  cnt0 = cnt0.reshape(N, 1)
  cnt1 = cnt1.reshape(N, 1)
  h1, st1 = _lin(aggs, cnt0, cnt1, xs, Wl1.T, r2(bl1), Wr1.T)
  xs = _bn(h1, st1, r2(g1), r2(beta1), split=True)

  aggs = _get_sc_agg(False)(*xs, graph3, z16, z1, ones128)
  h2, st2 = _lin(aggs, cnt0, cnt1, xs, Wl2.T, r2(bl2), Wr2.T)
  return _bn(h2, st2, r2(g2), r2(beta2), split=False)


# speedup vs baseline: 7.6318x; 1.4948x over previous
import functools

import jax
import jax.numpy as jnp
from jax import lax
from jax.experimental import pallas as pl
from jax.experimental.pallas import tpu as pltpu
from jax.experimental.pallas import tpu_sc as plsc

N_USERS = 10000
N_BOOKS = 40000
N = N_USERS + N_BOOKS
E = 800000
D_IN = 128
H = 64
Q = 16

NC = 2
NS = 16

ROWS = E // 128
FULL = 48
RQT = N // NS
CNT_Q = 3128

BLK = 2000
GRID = N // BLK
f32 = jnp.float32


def _make_sc_agg(with_cnt):
  mesh = plsc.VectorSubcoreMesh(
      core_axis_name="c", subcore_axis_name="s", num_cores=NC, num_subcores=NS)
  out_type = [jax.ShapeDtypeStruct((N, Q), f32) for _ in range(4)]
  if with_cnt:
    out_type += [jax.ShapeDtypeStruct((N,), f32),
                 jax.ShapeDtypeStruct((N,), f32)]
  scratch = [
      pltpu.VMEM((2, 8, 128), jnp.int32),
      pltpu.VMEM((2, 8, 128), jnp.int32),
      pltpu.VMEM((8, 128, Q), f32),
      pltpu.VMEM((8, 128, Q), f32),
      pltpu.VMEM((128,), f32),
      pltpu.VMEM_SHARED((N, Q), f32),
      pltpu.VMEM_SHARED((N,), f32),
      pltpu.SemaphoreType.DMA,
      pltpu.SemaphoreType.DMA,
      pltpu.SemaphoreType.DMA,
      pltpu.SemaphoreType.DMA,
      pltpu.SemaphoreType.DMA,
  ]
  CNT_L = N - (NS - 1) * CNT_Q

  def body(x00, x01, x10, x11, graph3, z16, z1, ones_hbm, *rest):
    aggs = rest[:4]
    rest = rest[4:]
    if with_cnt:
      cnt0, cnt1 = rest[:2]
      rest = rest[2:]
    idxA, idxB, rowsA, rowsB, ones_v, acc, cacc, gA, gB, sA, sB, sC = rest
    c = lax.axis_index("c")
    s = lax.axis_index("s")
    my0 = s * RQT
    row0 = s * 390 + jnp.minimum(s, 10)
    tail = jnp.where(s < 10, 7, 6)

    def zero_acc():
      pltpu.sync_copy(z16.at[pl.ds(my0, RQT), :], acc.at[pl.ds(my0, RQT), :])

    def cnt_slices(a, b):
      @pl.when(s < NS - 1)
      def _a():
        pltpu.sync_copy(a.at[pl.ds(s * CNT_Q, CNT_Q)],
                        b.at[pl.ds(s * CNT_Q, CNT_Q)])

      @pl.when(s == NS - 1)
      def _b():
        pltpu.sync_copy(a.at[pl.ds((NS - 1) * CNT_Q, CNT_L)],
                        b.at[pl.ds((NS - 1) * CNT_Q, CNT_L)])

    zero_acc()
    if with_cnt:
      cnt_slices(z1, cacc)
      pltpu.sync_copy(ones_hbm, ones_v)
    plsc.subcore_barrier()

    nrows = 384 + tail
    base48 = jnp.minimum(row0 + FULL * 8, ROWS - 8)

    def scatter_all(xq, parity, do_cnt):
      def stage_fire(idx, rws, gsem, base):
        pltpu.sync_copy(graph3.at[:, pl.ds(base, 8), :], idx)
        for j in range(8):
          pltpu.async_copy(xq.at[idx.at[0, j]], rws.at[j], gsem)

      def drain_gathers(idx, rws, gsem):
        for j in range(8):
          pltpu.make_async_copy(xq.at[idx.at[0, j]], rws.at[j], gsem).wait()

      def fire_scatters(idx, rws, sem):
        for j in range(8):
          pltpu.async_copy(rws.at[j], acc.at[idx.at[1, j]], sem, add=True)

      def drain_scatters(idx, rws, sem):
        for j in range(8):
          pltpu.make_async_copy(rws.at[j], acc.at[idx.at[1, j]], sem).wait()

      def cnt_fire(idx):
        for j in range(8):
          pltpu.async_copy(ones_v, cacc.at[idx.at[1, j]], sC, add=True)

      def cnt_drain(idx):
        for j in range(8):
          pltpu.make_async_copy(ones_v, cacc.at[idx.at[1, j]], sC).wait()

      stage_fire(idxA, rowsA, gA, row0)

      def step(kc, carry):
        def process(ci, cr, cg, cs, ni, nr, ng, ns_):
          @pl.when(kc > 0)
          def _drain_prev():
            drain_scatters(ni, nr, ns_)
            if do_cnt:
              @pl.when(((kc + 1) % 2) == parity)
              def _dc():
                cnt_drain(ni)
          stage_fire(ni, nr, ng, jnp.minimum(row0 + (kc + 1) * 8, ROWS - 8))
          drain_gathers(ci, cr, cg)
          fire_scatters(ci, cr, cs)
          if do_cnt:
            @pl.when((kc % 2) == parity)
            def _fc():
              cnt_fire(ci)

        @pl.when(kc % 2 == 0)
        def _even():
          process(idxA, rowsA, gA, sA, idxB, rowsB, gB, sB)

        @pl.when(kc % 2 == 1)
        def _odd():
          process(idxB, rowsB, gB, sB, idxA, rowsA, gA, sA)

        return carry

      lax.fori_loop(0, FULL, step, 0)

      drain_scatters(idxB, rowsB, sB)
      if do_cnt and parity == 1:
        cnt_drain(idxB)
      drain_gathers(idxA, rowsA, gA)
      for j in range(8):
        r = base48 + j

        @pl.when(jnp.logical_and(r >= row0 + FULL * 8, r < row0 + nrows))
        def _tail_scatter():
          pltpu.sync_copy(rowsA.at[j], acc.at[idxA.at[1, j]], add=True)
          if do_cnt and parity == 0:
            pltpu.sync_copy(ones_v, cacc.at[idxA.at[1, j]], add=True)

    for p in range(2):
      do_cnt = with_cnt and p == 0

      @pl.when(c == 0)
      def _run0():
        scatter_all(x00 if p == 0 else x01, 0, do_cnt)

      @pl.when(c == 1)
      def _run1():
        scatter_all(x10 if p == 0 else x11, 1, do_cnt)

      plsc.subcore_barrier()

      @pl.when(c == 0)
      def _out0():
        pltpu.sync_copy(acc.at[pl.ds(my0, RQT), :],
                        aggs[p].at[pl.ds(my0, RQT), :])

      @pl.when(c == 1)
      def _out1():
        pltpu.sync_copy(acc.at[pl.ds(my0, RQT), :],
                        aggs[2 + p].at[pl.ds(my0, RQT), :])

      if do_cnt:
        @pl.when(c == 0)
        def _outc0():
          cnt_slices(cacc, cnt0)

        @pl.when(c == 1)
        def _outc1():
          cnt_slices(cacc, cnt1)

      if p == 0:
        zero_acc()
        plsc.subcore_barrier()

  return pl.kernel(
      body, out_type=out_type, mesh=mesh, scratch_types=scratch,
      compiler_params=pltpu.CompilerParams(use_tc_tiling_on_sc=False))


@functools.lru_cache(maxsize=None)
def _get_sc_agg(with_cnt):
  return _make_sc_agg(with_cnt)


def _embed_body(uf, bf, w1u, b1u, w2u, b2u, w1b, b1b, w2b, b2b,
                o0, o1, o2, o3):
  is_user = pl.program_id(0) < N_USERS // BLK
  f = jnp.where(is_user, uf[...], bf[...])
  w1 = jnp.where(is_user, w1u[...], w1b[...])
  b1 = jnp.where(is_user, b1u[...], b1b[...])
  w2 = jnp.where(is_user, w2u[...], w2b[...])
  b2 = jnp.where(is_user, b2u[...], b2b[...])
  h = jnp.maximum(jnp.dot(f, w1, preferred_element_type=f32) + b1, 0.0)
  e = jnp.dot(h, w2, preferred_element_type=f32) + b2
  for k, o in enumerate((o0, o1, o2, o3)):
    o[...] = e[:, k * Q:(k + 1) * Q]


def _embed(uf, bf, w1ut, b1u, w2ut, b2u, w1bt, b1b, w2bt, b2b):
  cmap = lambda i: (0, 0)
  qspec = pl.BlockSpec((BLK, Q), lambda i: (i, 0))
  return pl.pallas_call(
      _embed_body,
      grid=(GRID,),
      in_specs=[
          pl.BlockSpec((BLK, D_IN), lambda i: (jnp.minimum(i, N_USERS // BLK - 1), 0)),
          pl.BlockSpec((BLK, D_IN), lambda i: (jnp.maximum(i - N_USERS // BLK, 0), 0)),
          pl.BlockSpec((D_IN, H), cmap), pl.BlockSpec((1, H), cmap),
          pl.BlockSpec((H, H), cmap), pl.BlockSpec((1, H), cmap),
          pl.BlockSpec((D_IN, H), cmap), pl.BlockSpec((1, H), cmap),
          pl.BlockSpec((H, H), cmap), pl.BlockSpec((1, H), cmap),
      ],
      out_specs=[qspec] * 4,
      out_shape=[jax.ShapeDtypeStruct((N, Q), f32)] * 4,
  )(uf, bf, w1ut, b1u, w2ut, b2u, w1bt, b1b, w2bt, b2b)


def _lin_body(a0, a1, a2, a3, c0, c1, x0, x1, x2, x3, wlt, bl, wrt,
              h_ref, st_ref, acc):
  i = pl.program_id(0)
  agg = jnp.concatenate([a0[...], a1[...], a2[...], a3[...]], axis=1)
  cnt = c0[...] + c1[...]
  rc = 1.0 / jnp.maximum(cnt, 1.0)
  x = jnp.concatenate([x0[...], x1[...], x2[...], x3[...]], axis=1)
  h = (jnp.dot(agg * rc, wlt[...], preferred_element_type=f32) + bl[...]
       + jnp.dot(x, wrt[...], preferred_element_type=f32))
  h_ref[...] = h
  st = jnp.stack([jnp.sum(h, axis=0), jnp.sum(h * h, axis=0)])

  @pl.when(i == 0)
  def _init():
    acc[...] = st

  @pl.when(i > 0)
  def _accum():
    acc[...] += st

  st_ref[...] = acc[...]


def _lin(aggs, cnt0, cnt1, xs, wlt, bl, wrt):
  cmap = lambda i: (0, 0)
  qspec = pl.BlockSpec((BLK, Q), lambda i: (i, 0))
  return pl.pallas_call(
      _lin_body,
      grid=(GRID,),
      in_specs=[qspec] * 4 + [
          pl.BlockSpec((BLK, 1), lambda i: (i, 0)),
          pl.BlockSpec((BLK, 1), lambda i: (i, 0)),
      ] + [qspec] * 4 + [
          pl.BlockSpec((H, H), cmap), pl.BlockSpec((1, H), cmap),
          pl.BlockSpec((H, H), cmap),
      ],
      out_specs=[pl.BlockSpec((BLK, H), lambda i: (i, 0)),
                 pl.BlockSpec((2, H), cmap)],
      out_shape=[jax.ShapeDtypeStruct((N, H), f32),
                 jax.ShapeDtypeStruct((2, H), f32)],
      scratch_shapes=[pltpu.VMEM((2, H), f32)],
  )(*aggs, cnt0, cnt1, *xs, wlt, bl, wrt)


def _bn_body_split(h_ref, st_ref, g_ref, b_ref, o0, o1, o2, o3):
  st = st_ref[...]
  m = st[0] / N
  v = st[1] / N - m * m
  sc = g_ref[0] * lax.rsqrt(v + 1e-5)
  y = jnp.maximum((h_ref[...] - m) * sc + b_ref[0], 0.0)
  for k, o in enumerate((o0, o1, o2, o3)):
    o[...] = y[:, k * Q:(k + 1) * Q]


def _bn_body_full(h_ref, st_ref, g_ref, b_ref, o_ref):
  st = st_ref[...]
  m = st[0] / N
  v = st[1] / N - m * m
  sc = g_ref[0] * lax.rsqrt(v + 1e-5)
  o_ref[...] = jnp.maximum((h_ref[...] - m) * sc + b_ref[0], 0.0)


def _bn(h, st, g, b, split):
  cmap = lambda i: (0, 0)
  in_specs = [
      pl.BlockSpec((BLK, H), lambda i: (i, 0)),
      pl.BlockSpec((2, H), cmap),
      pl.BlockSpec((1, H), cmap), pl.BlockSpec((1, H), cmap),
  ]
  if split:
    qspec = pl.BlockSpec((BLK, Q), lambda i: (i, 0))
    return pl.pallas_call(
        _bn_body_split, grid=(GRID,), in_specs=in_specs,
        out_specs=[qspec] * 4,
        out_shape=[jax.ShapeDtypeStruct((N, Q), f32)] * 4,
    )(h, st, g, b)
  return pl.pallas_call(
      _bn_body_full, grid=(GRID,), in_specs=in_specs,
      out_specs=pl.BlockSpec((BLK, H), lambda i: (i, 0)),
      out_shape=jax.ShapeDtypeStruct((N, H), f32),
  )(h, st, g, b)


def kernel(graph_data, user_features, book_features, W1u, b1u, W2u, b2u,
           W1b, b1b, W2b, b2b, Wl1, bl1, Wr1, Wl2, bl2, Wr2,
           g1, beta1, g2, beta2):
  graph3 = graph_data.reshape(2, ROWS, 128)

  z16 = jnp.zeros((N, Q), f32)
  z1 = jnp.zeros((N,), f32)
  ones128 = jnp.ones((128,), f32)

  r2 = lambda a: a.reshape(1, H)
  xs = _embed(user_features, book_features,
              W1u.T, r2(b1u), W2u.T, r2(b2u),
              W1b.T, r2(b1b), W2b.T, r2(b2b))

  *aggs, cnt0, cnt1 = _get_sc_agg(True)(*xs, graph3, z16, z1, ones128)
  cnt0 = cnt0.reshape(N, 1)
  cnt1 = cnt1.reshape(N, 1)
  h1, st1 = _lin(aggs, cnt0, cnt1, xs, Wl1.T, r2(bl1), Wr1.T)
  xs = _bn(h1, st1, r2(g1), r2(beta1), split=True)

  aggs = _get_sc_agg(False)(*xs, graph3, z16, z1, ones128)
  h2, st2 = _lin(aggs, cnt0, cnt1, xs, Wl2.T, r2(bl2), Wr2.T)
  return _bn(h2, st2, r2(g2), r2(beta2), split=False)
